# Initial kernel scaffold; baseline (speedup 1.0000x reference)
import jax
import jax.numpy as jnp
from jax.experimental import pallas as pl


def kernel(x, ygen_id, ygen, codebook, W1, b1, W2, b2, W3, b3, Wg, bg, Wrel, brel, Wroot, A1, a1, A2, a2, A3, a3, C1, c1, C2, c2, C3, c3):
    raise NotImplementedError("write your pallas kernel here")


# speedup vs baseline: 7.4651x; 7.4651x over previous
import functools

import jax
import jax.numpy as jnp
from jax import lax
from jax.experimental import pallas as pl
from jax.experimental.pallas import tpu as pltpu
from jax.experimental.pallas import tpu_sc as plsc

N = 5000
NP = 5120
BLK = 128
NBLK = NP // BLK
NBINS = 50
BIN = 100
K = 16
FH = 128
FO = 128

_NEG_SLOPE = 0.01


def _lrelu(v):
    return jnp.where(v >= 0, v, _NEG_SLOPE * v)


def _enc_body(x_ref, w1_ref, b1_ref, w2_ref, b2_ref, w3_ref, b3_ref, cb_ref,
              wg_ref, hxw_ref, bin_ref, rank_ref, off_ref, carry):
    i = pl.program_id(0)

    @pl.when(i == 0)
    def _init():
        carry[...] = jnp.zeros_like(carry)

    xb = x_ref[...]
    h1 = _lrelu(jnp.dot(xb, w1_ref[...]) + b1_ref[...])
    h2 = _lrelu(jnp.dot(h1, w2_ref[...]) + b2_ref[...])
    h = jnp.dot(h2, w3_ref[...]) + b3_ref[...]

    mul = jnp.dot(h, cb_ref[...])
    io25 = lax.broadcasted_iota(jnp.int32, (BLK, 25), 1)
    mx = jnp.maximum(jnp.max(mul, axis=1, keepdims=True),
                     jnp.max(-mul, axis=1, keepdims=True))
    bpos = jnp.min(jnp.where(mul == mx, io25, 64), axis=1, keepdims=True)
    bneg = jnp.min(jnp.where(-mul == mx, io25 + 25, 64), axis=1, keepdims=True)
    binv = jnp.minimum(bpos, bneg)
    rowid = i * BLK + lax.broadcasted_iota(jnp.int32, (BLK, 1), 0)
    binv = jnp.where(rowid < N, binv, NBINS)

    io64 = lax.broadcasted_iota(jnp.int32, (BLK, 64), 1)
    onehot = (io64 == binv).astype(jnp.float32)
    ior = lax.broadcasted_iota(jnp.int32, (BLK, BLK), 0)
    ioc = lax.broadcasted_iota(jnp.int32, (BLK, BLK), 1)
    lower = (ior > ioc).astype(jnp.float32)
    cum = jnp.dot(lower, onehot, precision=lax.Precision.HIGHEST)
    carry_now = carry[...]
    rank = jnp.sum(onehot * (carry_now + cum), axis=1, keepdims=True)
    carry[...] = carry_now + jnp.sum(onehot, axis=0, keepdims=True)

    hxw_ref[:, 0:12] = h
    hxw_ref[:, 12:16] = jnp.zeros((BLK, 4), jnp.float32)
    hxw_ref[:, 16:48] = jnp.dot(_lrelu(h), wg_ref[...])
    hxw_ref[:, 48:FH] = jnp.zeros((BLK, FH - 48), jnp.float32)
    bin_ref[...] = binv
    rank_ref[...] = rank.astype(jnp.int32)

    io64r = lax.broadcasted_iota(jnp.int32, (64, 64), 0)
    io64c = lax.broadcasted_iota(jnp.int32, (64, 64), 1)
    strict = (io64r < io64c).astype(jnp.float32)
    off_ref[...] = jnp.dot(carry[...], strict,
                           precision=lax.Precision.HIGHEST).astype(jnp.int32)


def _run_encoder(x_p, W1, b1, W2, b2, W3, b3, cb25, Wg):
    full = lambda s: pl.BlockSpec(s, lambda i: (0, 0))
    return pl.pallas_call(
        _enc_body,
        grid=(NBLK,),
        in_specs=[
            pl.BlockSpec((BLK, 12), lambda i: (i, 0)),
            full((12, 125)), full((1, 125)),
            full((125, 125)), full((1, 125)),
            full((125, 12)), full((1, 12)),
            full((12, 25)), full((12, 32)),
        ],
        out_specs=[
            pl.BlockSpec((BLK, FH), lambda i: (i, 0)),
            pl.BlockSpec((BLK, 1), lambda i: (i, 0)),
            pl.BlockSpec((BLK, 1), lambda i: (i, 0)),
            pl.BlockSpec((1, 64), lambda i: (0, 0)),
        ],
        out_shape=[
            jax.ShapeDtypeStruct((NP, FH), jnp.float32),
            jax.ShapeDtypeStruct((NP, 1), jnp.int32),
            jax.ShapeDtypeStruct((NP, 1), jnp.int32),
            jax.ShapeDtypeStruct((1, 64), jnp.int32),
        ],
        scratch_shapes=[pltpu.VMEM((1, 64), jnp.float32)],
    )(x_p, W1, b1, W2, b2, W3, b3, cb25, Wg)


def _pos_body(bin_ref, rank_ref, off_ref, pos_ref, posg_ref):
    binv = bin_ref[...]
    io64 = lax.broadcasted_iota(jnp.int32, (BLK, 64), 1)
    onehot = (io64 == binv).astype(jnp.float32)
    offsf = off_ref[...].astype(jnp.float32)
    posf = jnp.sum(onehot * offsf, axis=1, keepdims=True) \
        + rank_ref[...].astype(jnp.float32)
    pos = posf.astype(jnp.int32)
    pos_ref[...] = pos
    posg_ref[...] = jnp.minimum(pos, N - 1)


def _run_pos(binc, rankc, offs):
    return pl.pallas_call(
        _pos_body,
        grid=(NBLK,),
        in_specs=[
            pl.BlockSpec((BLK, 1), lambda i: (i, 0)),
            pl.BlockSpec((BLK, 1), lambda i: (i, 0)),
            pl.BlockSpec((1, 64), lambda i: (0, 0)),
        ],
        out_specs=[
            pl.BlockSpec((BLK, 1), lambda i: (i, 0)),
            pl.BlockSpec((BLK, 1), lambda i: (i, 0)),
        ],
        out_shape=[
            jax.ShapeDtypeStruct((NP, 1), jnp.int32),
            jax.ShapeDtypeStruct((NP, 1), jnp.int32),
        ],
        compiler_params=pltpu.CompilerParams(
            dimension_semantics=("arbitrary",)),
    )(binc, rankc, offs)


def _dotT(a, b):
    return lax.dot_general(a, b, (((0,), (0,)), ((), ())),
                           precision=lax.Precision.HIGHEST)


def _bin_body(ph_ref, wrel_ref, brel_ref, wroot_ref, bg_ref,
              a1w_ref, a1b_ref, a2w_ref, a2b_ref, a3w_ref, a3b_ref,
              c1a_ref, c1b_ref, c1bias_ref, c2w_ref, c2b_ref, c3w_ref, c3b_ref,
              out_ref):
    blk = ph_ref[0]
    pb = blk[:, 0:12]
    pxw = blk[:, 16:48]

    z = lax.dot_general(pb, pb, (((1,), (1,)), ((), ())))
    sim = jax.nn.sigmoid(z)

    io = lax.broadcasted_iota(jnp.int32, (BIN, BIN), 1)
    remain = jnp.ones((BIN, BIN), jnp.bool_)
    for _ in range(K):
        masked = jnp.where(remain, sim, -1.0)
        m = jnp.max(masked, axis=1, keepdims=True)
        cand = remain & (masked == m)
        jstar = jnp.min(jnp.where(cand, io, BIN), axis=1, keepdims=True)
        remain = remain & (io != jstar)
    adj = sim * (1.0 - remain.astype(jnp.float32))

    ones = jnp.ones((BIN, 1), jnp.float32)
    colsum = _dotT(adj, ones)
    dis = lax.rsqrt(colsum + 1.0)
    y = pxw * dis
    t1 = _dotT(adj, y) + y
    g = t1 * dis + bg_ref[...]
    aggr = _dotT(adj, g)
    g2 = jnp.dot(aggr, wrel_ref[...]) + brel_ref[...] + jnp.dot(g, wroot_ref[...])
    x2 = _lrelu(g2)

    c = _lrelu(jnp.dot(x2, a1w_ref[...]) + a1b_ref[...])
    c = _lrelu(jnp.dot(c, a2w_ref[...]) + a2b_ref[...])
    ids = jnp.dot(c, a3w_ref[...]) + a3b_ref[...]

    p = _lrelu(jnp.dot(x2, c1a_ref[...]) + jnp.dot(ids, c1b_ref[...]) + c1bias_ref[...])
    p = _lrelu(jnp.dot(p, c2w_ref[...]) + c2b_ref[...])
    p4 = jnp.dot(p, c3w_ref[...]) + c3b_ref[...]

    out_ref[0, :, 0:6] = ids
    out_ref[0, :, 6:12] = p4
    out_ref[0, :, 12:FO] = jnp.zeros((BIN, FO - 12), jnp.float32)


def _run_bins(phb, Wrel, brel, Wroot, bg, A1, a1, A2, a2, A3, a3,
              C1a, C1b, c1, C2, c2, C3, c3):
    full = lambda s: pl.BlockSpec(s, lambda i: (0, 0))
    return pl.pallas_call(
        _bin_body,
        grid=(NBINS,),
        in_specs=[
            pl.BlockSpec((1, BIN, FH), lambda i: (i, 0, 0)),
            full((32, 32)), full((1, 32)), full((32, 32)), full((1, 32)),
            full((32, 125)), full((1, 125)), full((125, 125)), full((1, 125)),
            full((125, 6)), full((1, 6)),
            full((32, 125)), full((6, 125)), full((1, 125)),
            full((125, 125)), full((1, 125)), full((125, 6)), full((1, 6)),
        ],
        out_specs=pl.BlockSpec((1, BIN, FO), lambda i: (i, 0, 0)),
        out_shape=jax.ShapeDtypeStruct((NBINS, BIN, FO), jnp.float32),
        compiler_params=pltpu.CompilerParams(
            dimension_semantics=("arbitrary",)),
    )(phb, Wrel, brel, Wroot, bg, A1, a1, A2, a2, A3, a3,
      C1a, C1b, c1, C2, c2, C3, c3)


_MESH = dict(core_axis_name="c", subcore_axis_name="s")
NW = 32
CH = NP // NW
HF = CH // 2


def _scatter_body(hxw, posh, phxw, pos0, pos1, rows0, rows1, sem):
    wid = lax.axis_index("s") * 2 + lax.axis_index("c")
    base = wid * CH
    for half, (posr, rowsr) in enumerate(((pos0, rows0), (pos1, rows1))):
        b2 = base + half * HF
        pltpu.sync_copy(posh.at[pl.ds(b2, HF)], posr)
        pltpu.sync_copy(hxw.at[pl.ds(b2, HF)], rowsr)
        pltpu.async_copy(rowsr, phxw.at[posr], sem).wait()


def _run_scatter(hxw, pos):
    kern = functools.partial(
        pl.kernel,
        mesh=plsc.VectorSubcoreMesh(**_MESH),
        out_type=jax.ShapeDtypeStruct((NP, FH), jnp.float32),
        scratch_types=[
            pltpu.VMEM((HF,), jnp.int32),
            pltpu.VMEM((HF,), jnp.int32),
            pltpu.VMEM((HF, FH), jnp.float32),
            pltpu.VMEM((HF, FH), jnp.float32),
            pltpu.SemaphoreType.DMA,
        ],
    )(_scatter_body)
    return kern(hxw, pos)


def _gather_body(pres, posh, outh, pos0, pos1, rows0, rows1, sem):
    wid = lax.axis_index("s") * 2 + lax.axis_index("c")
    base = wid * CH
    for half, (posr, rowsr) in enumerate(((pos0, rows0), (pos1, rows1))):
        b2 = base + half * HF
        pltpu.sync_copy(posh.at[pl.ds(b2, HF)], posr)
        pltpu.async_copy(pres.at[posr], rowsr, sem).wait()
        pltpu.sync_copy(rowsr, outh.at[pl.ds(b2, HF)])


def _run_gather(pres, pos):
    kern = functools.partial(
        pl.kernel,
        mesh=plsc.VectorSubcoreMesh(**_MESH),
        out_type=jax.ShapeDtypeStruct((NP, FO), jnp.float32),
        scratch_types=[
            pltpu.VMEM((HF,), jnp.int32),
            pltpu.VMEM((HF,), jnp.int32),
            pltpu.VMEM((HF, FO), jnp.float32),
            pltpu.VMEM((HF, FO), jnp.float32),
            pltpu.SemaphoreType.DMA,
        ],
    )(_gather_body)
    return kern(pres, pos)


def kernel(x, ygen_id, ygen, codebook, W1, b1, W2, b2, W3, b3, Wg, bg, Wrel,
           brel, Wroot, A1, a1, A2, a2, A3, a3, C1, c1, C2, c2, C3, c3):
    row = lambda v: v.reshape(1, -1)
    x_p = jnp.concatenate([x, jnp.zeros((NP - N, 12), x.dtype)], axis=0)

    hxw, binc, rankc, offs = _run_encoder(
        x_p, W1, row(b1), W2, row(b2), W3, row(b3), codebook[:, :25], Wg)

    posc, posgc = _run_pos(binc, rankc, offs)
    pos = posc.reshape(NP)
    posg = posgc.reshape(NP)
    phxw = _run_scatter(hxw, pos)

    phb = phxw[:N].reshape(NBINS, BIN, FH)
    pres = _run_bins(phb, Wrel, row(brel), Wroot, row(bg),
                     A1, row(a1), A2, row(a2), A3, row(a3),
                     C1[:32], C1[32:], row(c1), C2, row(c2), C3, row(c3))

    outp = _run_gather(pres.reshape(N, FO), posg)
    cand_ids = outp[:N, 0:6]
    cand_p4 = outp[:N, 6:12]
    return (cand_ids, cand_p4, ygen_id, ygen)


# speedup vs baseline: 10.3066x; 1.3806x over previous
import functools

import jax
import jax.numpy as jnp
from jax import lax
from jax.experimental import pallas as pl
from jax.experimental.pallas import tpu as pltpu
from jax.experimental.pallas import tpu_sc as plsc

N = 5000
NP = 5120
BLK = 128
NBLK = NP // BLK
NBINS = 50
BIN = 100
K = 16
FH = 128
FO = 128

_NEG_SLOPE = 0.01


def _lrelu(v):
    return jnp.where(v >= 0, v, _NEG_SLOPE * v)


def _enc_body(x_ref, w1_ref, b1_ref, w2_ref, b2_ref, w3_ref, b3_ref, cb_ref,
              wg_ref, hxw_ref, bin_ref, rank_ref, off_ref, carry):
    i = pl.program_id(0)

    @pl.when(i == 0)
    def _init():
        carry[...] = jnp.zeros_like(carry)

    xb = x_ref[...]
    h1 = _lrelu(jnp.dot(xb, w1_ref[...]) + b1_ref[...])
    h2 = _lrelu(jnp.dot(h1, w2_ref[...]) + b2_ref[...])
    h = jnp.dot(h2, w3_ref[...]) + b3_ref[...]

    mul = jnp.dot(h, cb_ref[...])
    io25 = lax.broadcasted_iota(jnp.int32, (BLK, 25), 1)
    mx = jnp.maximum(jnp.max(mul, axis=1, keepdims=True),
                     jnp.max(-mul, axis=1, keepdims=True))
    bpos = jnp.min(jnp.where(mul == mx, io25, 64), axis=1, keepdims=True)
    bneg = jnp.min(jnp.where(-mul == mx, io25 + 25, 64), axis=1, keepdims=True)
    binv = jnp.minimum(bpos, bneg)
    rowid = i * BLK + lax.broadcasted_iota(jnp.int32, (BLK, 1), 0)
    binv = jnp.where(rowid < N, binv, NBINS)

    io64 = lax.broadcasted_iota(jnp.int32, (BLK, 64), 1)
    onehot = (io64 == binv).astype(jnp.float32)
    ior = lax.broadcasted_iota(jnp.int32, (BLK, BLK), 0)
    ioc = lax.broadcasted_iota(jnp.int32, (BLK, BLK), 1)
    lower = (ior > ioc).astype(jnp.float32)
    cum = jnp.dot(lower, onehot, precision=lax.Precision.HIGHEST)
    carry_now = carry[...]
    rank = jnp.sum(onehot * (carry_now + cum), axis=1, keepdims=True)
    carry[...] = carry_now + jnp.sum(onehot, axis=0, keepdims=True)

    hxw_ref[:, 0:12] = h
    hxw_ref[:, 12:16] = jnp.zeros((BLK, 4), jnp.float32)
    hxw_ref[:, 16:48] = jnp.dot(_lrelu(h), wg_ref[...])
    hxw_ref[:, 48:FH] = jnp.zeros((BLK, FH - 48), jnp.float32)
    bin_ref[...] = binv
    rank_ref[...] = rank.astype(jnp.int32)

    io64r = lax.broadcasted_iota(jnp.int32, (64, 64), 0)
    io64c = lax.broadcasted_iota(jnp.int32, (64, 64), 1)
    strict = (io64r < io64c).astype(jnp.float32)
    off_ref[...] = jnp.dot(carry[...], strict,
                           precision=lax.Precision.HIGHEST).astype(jnp.int32)


def _run_encoder(x_p, W1, b1, W2, b2, W3, b3, cb25, Wg):
    full = lambda s: pl.BlockSpec(s, lambda i: (0, 0))
    return pl.pallas_call(
        _enc_body,
        grid=(NBLK,),
        in_specs=[
            pl.BlockSpec((BLK, 12), lambda i: (i, 0)),
            full((12, 125)), full((1, 125)),
            full((125, 125)), full((1, 125)),
            full((125, 12)), full((1, 12)),
            full((12, 25)), full((12, 32)),
        ],
        out_specs=[
            pl.BlockSpec((BLK, FH), lambda i: (i, 0)),
            pl.BlockSpec((BLK, 1), lambda i: (i, 0)),
            pl.BlockSpec((BLK, 1), lambda i: (i, 0)),
            pl.BlockSpec((1, 64), lambda i: (0, 0)),
        ],
        out_shape=[
            jax.ShapeDtypeStruct((NP, FH), jnp.float32),
            jax.ShapeDtypeStruct((NP, 1), jnp.int32),
            jax.ShapeDtypeStruct((NP, 1), jnp.int32),
            jax.ShapeDtypeStruct((1, 64), jnp.int32),
        ],
        scratch_shapes=[pltpu.VMEM((1, 64), jnp.float32)],
    )(x_p, W1, b1, W2, b2, W3, b3, cb25, Wg)


def _pos_body(bin_ref, rank_ref, off_ref, pos_ref, posg_ref):
    binv = bin_ref[...]
    io64 = lax.broadcasted_iota(jnp.int32, (BLK, 64), 1)
    onehot = (io64 == binv).astype(jnp.float32)
    offsf = off_ref[...].astype(jnp.float32)
    posf = jnp.sum(onehot * offsf, axis=1, keepdims=True) \
        + rank_ref[...].astype(jnp.float32)
    pos = posf.astype(jnp.int32)
    pos_ref[...] = pos
    posg_ref[...] = jnp.minimum(pos, N - 1)


def _run_pos(binc, rankc, offs):
    return pl.pallas_call(
        _pos_body,
        grid=(NBLK,),
        in_specs=[
            pl.BlockSpec((BLK, 1), lambda i: (i, 0)),
            pl.BlockSpec((BLK, 1), lambda i: (i, 0)),
            pl.BlockSpec((1, 64), lambda i: (0, 0)),
        ],
        out_specs=[
            pl.BlockSpec((BLK, 1), lambda i: (i, 0)),
            pl.BlockSpec((BLK, 1), lambda i: (i, 0)),
        ],
        out_shape=[
            jax.ShapeDtypeStruct((NP, 1), jnp.int32),
            jax.ShapeDtypeStruct((NP, 1), jnp.int32),
        ],
        compiler_params=pltpu.CompilerParams(
            dimension_semantics=("arbitrary",)),
    )(binc, rankc, offs)


def _dotT(a, b):
    return lax.dot_general(a, b, (((0,), (0,)), ((), ())),
                           precision=lax.Precision.HIGHEST)


BPER = 5


def _bin_body(ph_ref, wrel_ref, brel_ref, wroot_ref, bg_ref,
              a1w_ref, a1b_ref, a2w_ref, a2b_ref, a3w_ref, a3b_ref,
              c1a_ref, c1b_ref, c1bias_ref, c2w_ref, c2b_ref, c3w_ref, c3b_ref,
              out_ref):
    io = lax.broadcasted_iota(jnp.int32, (BIN, BIN), 1)
    sims = []
    for b in range(BPER):
        pb = ph_ref[b, :, 0:12]
        z = lax.dot_general(pb, pb, (((1,), (1,)), ((), ())))
        sims.append(jax.nn.sigmoid(z))

    live = list(sims)
    for _ in range(K):
        for b in range(BPER):
            m = jnp.max(live[b], axis=1, keepdims=True)
            cand = live[b] == m
            jstar = jnp.min(jnp.where(cand, io, BIN), axis=1, keepdims=True)
            live[b] = jnp.where(io == jstar, -1.0, live[b])

    for b in range(BPER):
        adj = jnp.where(live[b] < 0, sims[b], 0.0)
        pxw = ph_ref[b, :, 16:48]
        ones = jnp.ones((BIN, 1), jnp.float32)
        colsum = _dotT(adj, ones)
        dis = lax.rsqrt(colsum + 1.0)
        y = pxw * dis
        t1 = _dotT(adj, y) + y
        g = t1 * dis + bg_ref[...]
        aggr = _dotT(adj, g)
        g2 = jnp.dot(aggr, wrel_ref[...]) + brel_ref[...] + jnp.dot(g, wroot_ref[...])
        x2 = _lrelu(g2)

        c = _lrelu(jnp.dot(x2, a1w_ref[...]) + a1b_ref[...])
        c = _lrelu(jnp.dot(c, a2w_ref[...]) + a2b_ref[...])
        ids = jnp.dot(c, a3w_ref[...]) + a3b_ref[...]

        p = _lrelu(jnp.dot(x2, c1a_ref[...]) + jnp.dot(ids, c1b_ref[...])
                   + c1bias_ref[...])
        p = _lrelu(jnp.dot(p, c2w_ref[...]) + c2b_ref[...])
        p4 = jnp.dot(p, c3w_ref[...]) + c3b_ref[...]

        out_ref[b, :, 0:6] = ids
        out_ref[b, :, 6:12] = p4
        out_ref[b, :, 12:FO] = jnp.zeros((BIN, FO - 12), jnp.float32)


def _run_bins(phb, Wrel, brel, Wroot, bg, A1, a1, A2, a2, A3, a3,
              C1a, C1b, c1, C2, c2, C3, c3):
    full = lambda s: pl.BlockSpec(s, lambda i: (0, 0))
    return pl.pallas_call(
        _bin_body,
        grid=(NBINS // BPER,),
        in_specs=[
            pl.BlockSpec((BPER, BIN, FH), lambda i: (i, 0, 0)),
            full((32, 32)), full((1, 32)), full((32, 32)), full((1, 32)),
            full((32, 125)), full((1, 125)), full((125, 125)), full((1, 125)),
            full((125, 6)), full((1, 6)),
            full((32, 125)), full((6, 125)), full((1, 125)),
            full((125, 125)), full((1, 125)), full((125, 6)), full((1, 6)),
        ],
        out_specs=pl.BlockSpec((BPER, BIN, FO), lambda i: (i, 0, 0)),
        out_shape=jax.ShapeDtypeStruct((NBINS, BIN, FO), jnp.float32),
        compiler_params=pltpu.CompilerParams(
            dimension_semantics=("arbitrary",)),
    )(phb, Wrel, brel, Wroot, bg, A1, a1, A2, a2, A3, a3,
      C1a, C1b, c1, C2, c2, C3, c3)


_MESH = dict(core_axis_name="c", subcore_axis_name="s")
NW = 32
CH = NP // NW
HF = CH // 2


def _scatter_body(hxw, posh, phxw, pos0, pos1, rows0, rows1, sem):
    wid = lax.axis_index("s") * 2 + lax.axis_index("c")
    base = wid * CH
    for half, (posr, rowsr) in enumerate(((pos0, rows0), (pos1, rows1))):
        b2 = base + half * HF
        pltpu.sync_copy(posh.at[pl.ds(b2, HF)], posr)
        pltpu.sync_copy(hxw.at[pl.ds(b2, HF)], rowsr)
        pltpu.async_copy(rowsr, phxw.at[posr], sem).wait()


def _run_scatter(hxw, pos):
    kern = functools.partial(
        pl.kernel,
        mesh=plsc.VectorSubcoreMesh(**_MESH),
        out_type=jax.ShapeDtypeStruct((NP, FH), jnp.float32),
        scratch_types=[
            pltpu.VMEM((HF,), jnp.int32),
            pltpu.VMEM((HF,), jnp.int32),
            pltpu.VMEM((HF, FH), jnp.float32),
            pltpu.VMEM((HF, FH), jnp.float32),
            pltpu.SemaphoreType.DMA,
        ],
    )(_scatter_body)
    return kern(hxw, pos)


def _gather_body(pres, posh, outh, pos0, pos1, rows0, rows1, sem):
    wid = lax.axis_index("s") * 2 + lax.axis_index("c")
    base = wid * CH
    for half, (posr, rowsr) in enumerate(((pos0, rows0), (pos1, rows1))):
        b2 = base + half * HF
        pltpu.sync_copy(posh.at[pl.ds(b2, HF)], posr)
        pltpu.async_copy(pres.at[posr], rowsr, sem).wait()
        pltpu.sync_copy(rowsr, outh.at[pl.ds(b2, HF)])


def _run_gather(pres, pos):
    kern = functools.partial(
        pl.kernel,
        mesh=plsc.VectorSubcoreMesh(**_MESH),
        out_type=jax.ShapeDtypeStruct((NP, FO), jnp.float32),
        scratch_types=[
            pltpu.VMEM((HF,), jnp.int32),
            pltpu.VMEM((HF,), jnp.int32),
            pltpu.VMEM((HF, FO), jnp.float32),
            pltpu.VMEM((HF, FO), jnp.float32),
            pltpu.SemaphoreType.DMA,
        ],
    )(_gather_body)
    return kern(pres, pos)


def kernel(x, ygen_id, ygen, codebook, W1, b1, W2, b2, W3, b3, Wg, bg, Wrel,
           brel, Wroot, A1, a1, A2, a2, A3, a3, C1, c1, C2, c2, C3, c3):
    row = lambda v: v.reshape(1, -1)
    x_p = jnp.concatenate([x, jnp.zeros((NP - N, 12), x.dtype)], axis=0)

    hxw, binc, rankc, offs = _run_encoder(
        x_p, W1, row(b1), W2, row(b2), W3, row(b3), codebook[:, :25], Wg)

    posc, posgc = _run_pos(binc, rankc, offs)
    pos = posc.reshape(NP)
    posg = posgc.reshape(NP)
    phxw = _run_scatter(hxw, pos)

    phb = phxw[:N].reshape(NBINS, BIN, FH)
    pres = _run_bins(phb, Wrel, row(brel), Wroot, row(bg),
                     A1, row(a1), A2, row(a2), A3, row(a3),
                     C1[:32], C1[32:], row(c1), C2, row(c2), C3, row(c3))

    outp = _run_gather(pres.reshape(N, FO), posg)
    cand_ids = outp[:N, 0:6]
    cand_p4 = outp[:N, 6:12]
    return (cand_ids, cand_p4, ygen_id, ygen)


# speedup vs baseline: 12.8826x; 1.2499x over previous
import functools

import jax
import jax.numpy as jnp
from jax import lax
from jax.experimental import pallas as pl
from jax.experimental.pallas import tpu as pltpu
from jax.experimental.pallas import tpu_sc as plsc

N = 5000
NP = 5120
BLK = 256
NBLK = NP // BLK
NBINS = 50
BIN = 100
K = 16
FH = 128
FO = 128

_NEG_SLOPE = 0.01


def _lrelu(v):
    return jnp.where(v >= 0, v, _NEG_SLOPE * v)


def _enc_body(x_ref, w1_ref, b1_ref, w2_ref, b2_ref, w3_ref, b3_ref, cb_ref,
              wg_ref, hxw_ref, bin_ref, rank_ref, off_ref, carry):
    i = pl.program_id(0)

    @pl.when(i == 0)
    def _init():
        carry[...] = jnp.zeros_like(carry)

    xb = x_ref[...]
    h1 = _lrelu(jnp.dot(xb, w1_ref[...]) + b1_ref[...])
    h2 = _lrelu(jnp.dot(h1, w2_ref[...]) + b2_ref[...])
    h = jnp.dot(h2, w3_ref[...]) + b3_ref[...]

    mul = jnp.dot(h, cb_ref[...])
    io25 = lax.broadcasted_iota(jnp.int32, (BLK, 25), 1)
    mx = jnp.maximum(jnp.max(mul, axis=1, keepdims=True),
                     jnp.max(-mul, axis=1, keepdims=True))
    bpos = jnp.min(jnp.where(mul == mx, io25, 64), axis=1, keepdims=True)
    bneg = jnp.min(jnp.where(-mul == mx, io25 + 25, 64), axis=1, keepdims=True)
    binv = jnp.minimum(bpos, bneg)
    rowid = i * BLK + lax.broadcasted_iota(jnp.int32, (BLK, 1), 0)
    binv = jnp.where(rowid < N, binv, NBINS)

    io64 = lax.broadcasted_iota(jnp.int32, (BLK, 64), 1)
    onehot = (io64 == binv).astype(jnp.float32)
    ior = lax.broadcasted_iota(jnp.int32, (BLK, BLK), 0)
    ioc = lax.broadcasted_iota(jnp.int32, (BLK, BLK), 1)
    lower = (ior > ioc).astype(jnp.float32)
    cum = jnp.dot(lower, onehot, precision=lax.Precision.HIGHEST)
    carry_now = carry[...]
    rank = jnp.sum(onehot * (carry_now + cum), axis=1, keepdims=True)
    carry[...] = carry_now + jnp.sum(onehot, axis=0, keepdims=True)

    hxw_ref[:, 0:12] = h
    hxw_ref[:, 12:16] = jnp.zeros((BLK, 4), jnp.float32)
    hxw_ref[:, 16:48] = jnp.dot(_lrelu(h), wg_ref[...])
    hxw_ref[:, 48:FH] = jnp.zeros((BLK, FH - 48), jnp.float32)
    bin_ref[...] = binv
    rank_ref[...] = rank.astype(jnp.int32)

    io64r = lax.broadcasted_iota(jnp.int32, (64, 64), 0)
    io64c = lax.broadcasted_iota(jnp.int32, (64, 64), 1)
    strict = (io64r < io64c).astype(jnp.float32)
    off_ref[...] = jnp.dot(carry[...], strict,
                           precision=lax.Precision.HIGHEST).astype(jnp.int32)


def _run_encoder(x_p, W1, b1, W2, b2, W3, b3, cb25, Wg):
    full = lambda s: pl.BlockSpec(s, lambda i: (0, 0))
    return pl.pallas_call(
        _enc_body,
        grid=(NBLK,),
        in_specs=[
            pl.BlockSpec((BLK, 12), lambda i: (i, 0)),
            full((12, 125)), full((1, 125)),
            full((125, 125)), full((1, 125)),
            full((125, 12)), full((1, 12)),
            full((12, 25)), full((12, 32)),
        ],
        out_specs=[
            pl.BlockSpec((BLK, FH), lambda i: (i, 0)),
            pl.BlockSpec((BLK, 1), lambda i: (i, 0)),
            pl.BlockSpec((BLK, 1), lambda i: (i, 0)),
            pl.BlockSpec((1, 64), lambda i: (0, 0)),
        ],
        out_shape=[
            jax.ShapeDtypeStruct((NP, FH), jnp.float32),
            jax.ShapeDtypeStruct((NP, 1), jnp.int32),
            jax.ShapeDtypeStruct((NP, 1), jnp.int32),
            jax.ShapeDtypeStruct((1, 64), jnp.int32),
        ],
        scratch_shapes=[pltpu.VMEM((1, 64), jnp.float32)],
    )(x_p, W1, b1, W2, b2, W3, b3, cb25, Wg)


def _pos_body(bin_ref, rank_ref, off_ref, pos_ref, posg_ref):
    binv = bin_ref[...]
    io64 = lax.broadcasted_iota(jnp.int32, (BLK, 64), 1)
    onehot = (io64 == binv).astype(jnp.float32)
    offsf = off_ref[...].astype(jnp.float32)
    posf = jnp.sum(onehot * offsf, axis=1, keepdims=True) \
        + rank_ref[...].astype(jnp.float32)
    pos = posf.astype(jnp.int32)
    pos_ref[...] = pos
    posg_ref[...] = jnp.minimum(pos, N - 1)


def _run_pos(binc, rankc, offs):
    return pl.pallas_call(
        _pos_body,
        grid=(NBLK,),
        in_specs=[
            pl.BlockSpec((BLK, 1), lambda i: (i, 0)),
            pl.BlockSpec((BLK, 1), lambda i: (i, 0)),
            pl.BlockSpec((1, 64), lambda i: (0, 0)),
        ],
        out_specs=[
            pl.BlockSpec((BLK, 1), lambda i: (i, 0)),
            pl.BlockSpec((BLK, 1), lambda i: (i, 0)),
        ],
        out_shape=[
            jax.ShapeDtypeStruct((NP, 1), jnp.int32),
            jax.ShapeDtypeStruct((NP, 1), jnp.int32),
        ],
        compiler_params=pltpu.CompilerParams(
            dimension_semantics=("arbitrary",)),
    )(binc, rankc, offs)


def _dotT(a, b):
    return lax.dot_general(a, b, (((0,), (0,)), ((), ())),
                           precision=lax.Precision.HIGHEST)


BPER = 10


def _bin_body(ph_ref, wrel_ref, brel_ref, wroot_ref, bg_ref,
              a1w_ref, a1b_ref, a2w_ref, a2b_ref, a3w_ref, a3b_ref,
              c1a_ref, c1b_ref, c1bias_ref, c2w_ref, c2b_ref, c3w_ref, c3b_ref,
              out_ref):
    iof = lax.broadcasted_iota(jnp.int32, (BIN, BIN), 1).astype(jnp.float32)
    sims = []
    for b in range(BPER):
        pb = ph_ref[b, :, 0:12]
        z = lax.dot_general(pb, pb, (((1,), (1,)), ((), ())))
        sims.append(jax.nn.sigmoid(z))

    live = list(sims)
    for _ in range(K):
        for b in range(BPER):
            m = jnp.max(live[b], axis=1, keepdims=True)
            cand = live[b] == m
            jstar = jnp.min(jnp.where(cand, iof, float(BIN)),
                            axis=1, keepdims=True)
            live[b] = jnp.where(iof == jstar, -1.0, live[b])

    for b in range(BPER):
        adj = jnp.where(live[b] < 0, sims[b], 0.0)
        pxw = ph_ref[b, :, 16:48]
        ones = jnp.ones((BIN, 1), jnp.float32)
        colsum = _dotT(adj, ones)
        dis = lax.rsqrt(colsum + 1.0)
        y = pxw * dis
        t1 = _dotT(adj, y) + y
        g = t1 * dis + bg_ref[...]
        aggr = _dotT(adj, g)
        g2 = jnp.dot(aggr, wrel_ref[...]) + brel_ref[...] + jnp.dot(g, wroot_ref[...])
        x2 = _lrelu(g2)

        c = _lrelu(jnp.dot(x2, a1w_ref[...]) + a1b_ref[...])
        c = _lrelu(jnp.dot(c, a2w_ref[...]) + a2b_ref[...])
        ids = jnp.dot(c, a3w_ref[...]) + a3b_ref[...]

        p = _lrelu(jnp.dot(x2, c1a_ref[...]) + jnp.dot(ids, c1b_ref[...])
                   + c1bias_ref[...])
        p = _lrelu(jnp.dot(p, c2w_ref[...]) + c2b_ref[...])
        p4 = jnp.dot(p, c3w_ref[...]) + c3b_ref[...]

        out_ref[b, :, 0:6] = ids
        out_ref[b, :, 6:12] = p4
        out_ref[b, :, 12:FO] = jnp.zeros((BIN, FO - 12), jnp.float32)


def _run_bins(phb, Wrel, brel, Wroot, bg, A1, a1, A2, a2, A3, a3,
              C1a, C1b, c1, C2, c2, C3, c3):
    full = lambda s: pl.BlockSpec(s, lambda i: (0, 0))
    return pl.pallas_call(
        _bin_body,
        grid=(NBINS // BPER,),
        in_specs=[
            pl.BlockSpec((BPER, BIN, FH), lambda i: (i, 0, 0)),
            full((32, 32)), full((1, 32)), full((32, 32)), full((1, 32)),
            full((32, 125)), full((1, 125)), full((125, 125)), full((1, 125)),
            full((125, 6)), full((1, 6)),
            full((32, 125)), full((6, 125)), full((1, 125)),
            full((125, 125)), full((1, 125)), full((125, 6)), full((1, 6)),
        ],
        out_specs=pl.BlockSpec((BPER, BIN, FO), lambda i: (i, 0, 0)),
        out_shape=jax.ShapeDtypeStruct((NBINS, BIN, FO), jnp.float32),
        compiler_params=pltpu.CompilerParams(
            dimension_semantics=("arbitrary",)),
    )(phb, Wrel, brel, Wroot, bg, A1, a1, A2, a2, A3, a3,
      C1a, C1b, c1, C2, c2, C3, c3)


_MESH = dict(core_axis_name="c", subcore_axis_name="s")
NW = 32
CH = NP // NW
HF = CH // 2


def _scatter_body(hxw, posh, phxw, pos0, pos1, rows0, rows1, sem):
    wid = lax.axis_index("s") * 2 + lax.axis_index("c")
    base = wid * CH
    for half, (posr, rowsr) in enumerate(((pos0, rows0), (pos1, rows1))):
        b2 = base + half * HF
        pltpu.sync_copy(posh.at[pl.ds(b2, HF)], posr)
        pltpu.sync_copy(hxw.at[pl.ds(b2, HF)], rowsr)
        pltpu.async_copy(rowsr, phxw.at[posr], sem).wait()


def _run_scatter(hxw, pos):
    kern = functools.partial(
        pl.kernel,
        mesh=plsc.VectorSubcoreMesh(**_MESH),
        out_type=jax.ShapeDtypeStruct((NP, FH), jnp.float32),
        scratch_types=[
            pltpu.VMEM((HF,), jnp.int32),
            pltpu.VMEM((HF,), jnp.int32),
            pltpu.VMEM((HF, FH), jnp.float32),
            pltpu.VMEM((HF, FH), jnp.float32),
            pltpu.SemaphoreType.DMA,
        ],
    )(_scatter_body)
    return kern(hxw, pos)


def _gather_body(pres, posh, outh, pos0, pos1, rows0, rows1, sem):
    wid = lax.axis_index("s") * 2 + lax.axis_index("c")
    base = wid * CH
    for half, (posr, rowsr) in enumerate(((pos0, rows0), (pos1, rows1))):
        b2 = base + half * HF
        pltpu.sync_copy(posh.at[pl.ds(b2, HF)], posr)
        pltpu.async_copy(pres.at[posr], rowsr, sem).wait()
        pltpu.sync_copy(rowsr, outh.at[pl.ds(b2, HF)])


def _run_gather(pres, pos):
    kern = functools.partial(
        pl.kernel,
        mesh=plsc.VectorSubcoreMesh(**_MESH),
        out_type=jax.ShapeDtypeStruct((NP, FO), jnp.float32),
        scratch_types=[
            pltpu.VMEM((HF,), jnp.int32),
            pltpu.VMEM((HF,), jnp.int32),
            pltpu.VMEM((HF, FO), jnp.float32),
            pltpu.VMEM((HF, FO), jnp.float32),
            pltpu.SemaphoreType.DMA,
        ],
    )(_gather_body)
    return kern(pres, pos)


def kernel(x, ygen_id, ygen, codebook, W1, b1, W2, b2, W3, b3, Wg, bg, Wrel,
           brel, Wroot, A1, a1, A2, a2, A3, a3, C1, c1, C2, c2, C3, c3):
    row = lambda v: v.reshape(1, -1)
    x_p = jnp.concatenate([x, jnp.zeros((NP - N, 12), x.dtype)], axis=0)

    hxw, binc, rankc, offs = _run_encoder(
        x_p, W1, row(b1), W2, row(b2), W3, row(b3), codebook[:, :25], Wg)

    posc, posgc = _run_pos(binc, rankc, offs)
    pos = posc.reshape(NP)
    posg = posgc.reshape(NP)
    phxw = _run_scatter(hxw, pos)

    phb = phxw[:N].reshape(NBINS, BIN, FH)
    pres = _run_bins(phb, Wrel, row(brel), Wroot, row(bg),
                     A1, row(a1), A2, row(a2), A3, row(a3),
                     C1[:32], C1[32:], row(c1), C2, row(c2), C3, row(c3))

    outp = _run_gather(pres.reshape(N, FO), posg)
    cand_ids = outp[:N, 0:6]
    cand_p4 = outp[:N, 6:12]
    return (cand_ids, cand_p4, ygen_id, ygen)


# speedup vs baseline: 16.2453x; 1.2610x over previous
import functools

import jax
import jax.numpy as jnp
from jax import lax
from jax.experimental import pallas as pl
from jax.experimental.pallas import tpu as pltpu
from jax.experimental.pallas import tpu_sc as plsc

N = 5000
NP = 5120
BLK = 256
NBLK = NP // BLK
NBINS = 50
BIN = 100
K = 16
FH = 128
FO = 128

_NEG_SLOPE = 0.01


def _lrelu(v):
    return jnp.where(v >= 0, v, _NEG_SLOPE * v)


def _enc_body(x_ref, w1_ref, b1_ref, w2_ref, b2_ref, w3_ref, b3_ref, cb_ref,
              wg_ref, hxw_ref, bin_ref, rank_ref, off_ref, carry):
    i = pl.program_id(0)

    @pl.when(i == 0)
    def _init():
        carry[...] = jnp.zeros_like(carry)

    xb = x_ref[...]
    h1 = _lrelu(jnp.dot(xb, w1_ref[...]) + b1_ref[...])
    h2 = _lrelu(jnp.dot(h1, w2_ref[...]) + b2_ref[...])
    h = jnp.dot(h2, w3_ref[...]) + b3_ref[...]

    mul = jnp.dot(h, cb_ref[...])
    io25 = lax.broadcasted_iota(jnp.int32, (BLK, 25), 1)
    mx = jnp.maximum(jnp.max(mul, axis=1, keepdims=True),
                     jnp.max(-mul, axis=1, keepdims=True))
    bpos = jnp.min(jnp.where(mul == mx, io25, 64), axis=1, keepdims=True)
    bneg = jnp.min(jnp.where(-mul == mx, io25 + 25, 64), axis=1, keepdims=True)
    binv = jnp.minimum(bpos, bneg)
    rowid = i * BLK + lax.broadcasted_iota(jnp.int32, (BLK, 1), 0)
    binv = jnp.where(rowid < N, binv, NBINS)

    io64 = lax.broadcasted_iota(jnp.int32, (BLK, 64), 1)
    onehot = (io64 == binv).astype(jnp.float32)
    ior = lax.broadcasted_iota(jnp.int32, (BLK, BLK), 0)
    ioc = lax.broadcasted_iota(jnp.int32, (BLK, BLK), 1)
    lower = (ior > ioc).astype(jnp.float32)
    cum = jnp.dot(lower, onehot, precision=lax.Precision.HIGHEST)
    carry_now = carry[...]
    rank = jnp.sum(onehot * (carry_now + cum), axis=1, keepdims=True)
    carry[...] = carry_now + jnp.sum(onehot, axis=0, keepdims=True)

    hxw_ref[:, 0:12] = h
    hxw_ref[:, 12:16] = jnp.zeros((BLK, 4), jnp.float32)
    hxw_ref[:, 16:48] = jnp.dot(_lrelu(h), wg_ref[...])
    hxw_ref[:, 48:FH] = jnp.zeros((BLK, FH - 48), jnp.float32)
    bin_ref[...] = binv
    rank_ref[...] = rank.astype(jnp.int32)

    io64r = lax.broadcasted_iota(jnp.int32, (64, 64), 0)
    io64c = lax.broadcasted_iota(jnp.int32, (64, 64), 1)
    strict = (io64r < io64c).astype(jnp.float32)
    off_ref[...] = jnp.dot(carry[...], strict,
                           precision=lax.Precision.HIGHEST).astype(jnp.int32)


def _run_encoder(x_p, W1, b1, W2, b2, W3, b3, cb25, Wg):
    full = lambda s: pl.BlockSpec(s, lambda i: (0, 0))
    return pl.pallas_call(
        _enc_body,
        grid=(NBLK,),
        in_specs=[
            pl.BlockSpec((BLK, 12), lambda i: (i, 0)),
            full((12, 125)), full((1, 125)),
            full((125, 125)), full((1, 125)),
            full((125, 12)), full((1, 12)),
            full((12, 25)), full((12, 32)),
        ],
        out_specs=[
            pl.BlockSpec((BLK, FH), lambda i: (i, 0)),
            pl.BlockSpec((BLK, 1), lambda i: (i, 0)),
            pl.BlockSpec((BLK, 1), lambda i: (i, 0)),
            pl.BlockSpec((1, 64), lambda i: (0, 0)),
        ],
        out_shape=[
            jax.ShapeDtypeStruct((NP, FH), jnp.float32),
            jax.ShapeDtypeStruct((NP, 1), jnp.int32),
            jax.ShapeDtypeStruct((NP, 1), jnp.int32),
            jax.ShapeDtypeStruct((1, 64), jnp.int32),
        ],
        scratch_shapes=[pltpu.VMEM((1, 64), jnp.float32)],
    )(x_p, W1, b1, W2, b2, W3, b3, cb25, Wg)


def _pos_body(bin_ref, rank_ref, off_ref, pos_ref, posg_ref):
    binv = bin_ref[...]
    io64 = lax.broadcasted_iota(jnp.int32, (BLK, 64), 1)
    onehot = (io64 == binv).astype(jnp.float32)
    offsf = off_ref[...].astype(jnp.float32)
    posf = jnp.sum(onehot * offsf, axis=1, keepdims=True) \
        + rank_ref[...].astype(jnp.float32)
    pos = posf.astype(jnp.int32)
    pos_ref[...] = pos
    posg_ref[...] = jnp.minimum(pos, N - 1)


def _run_pos(binc, rankc, offs):
    return pl.pallas_call(
        _pos_body,
        grid=(NBLK,),
        in_specs=[
            pl.BlockSpec((BLK, 1), lambda i: (i, 0)),
            pl.BlockSpec((BLK, 1), lambda i: (i, 0)),
            pl.BlockSpec((1, 64), lambda i: (0, 0)),
        ],
        out_specs=[
            pl.BlockSpec((BLK, 1), lambda i: (i, 0)),
            pl.BlockSpec((BLK, 1), lambda i: (i, 0)),
        ],
        out_shape=[
            jax.ShapeDtypeStruct((NP, 1), jnp.int32),
            jax.ShapeDtypeStruct((NP, 1), jnp.int32),
        ],
        compiler_params=pltpu.CompilerParams(
            dimension_semantics=("arbitrary",)),
    )(binc, rankc, offs)


def _dotT(a, b):
    return lax.dot_general(a, b, (((0,), (0,)), ((), ())),
                           precision=lax.Precision.HIGHEST)


BPER = 10


def _bin_body(ph_ref, wrel_ref, brel_ref, wroot_ref, bg_ref,
              a1w_ref, a1b_ref, a2w_ref, a2b_ref, a3w_ref, a3b_ref,
              c1a_ref, c1b_ref, c1bias_ref, c2w_ref, c2b_ref, c3w_ref, c3b_ref,
              out_ref):
    io = lax.broadcasted_iota(jnp.int32, (BIN, BIN), 1)
    lives, adjs = [], []
    for b in range(BPER):
        pb = ph_ref[b, :, 0:12]
        z = lax.dot_general(pb, pb, (((1,), (1,)), ((), ())))
        lives.append(jax.nn.sigmoid(z))
        adjs.append(jnp.zeros((BIN, BIN), jnp.float32))

    for _ in range(K):
        for b in range(BPER):
            jidx = jnp.argmax(lives[b], axis=1).astype(jnp.int32)
            hit = io == jidx[:, None]
            adjs[b] = jnp.where(hit, lives[b], adjs[b])
            lives[b] = jnp.where(hit, -1.0, lives[b])

    for b in range(BPER):
        adj = adjs[b]
        pxw = ph_ref[b, :, 16:48]
        ones = jnp.ones((BIN, 1), jnp.float32)
        colsum = _dotT(adj, ones)
        dis = lax.rsqrt(colsum + 1.0)
        y = pxw * dis
        t1 = _dotT(adj, y) + y
        g = t1 * dis + bg_ref[...]
        aggr = _dotT(adj, g)
        g2 = jnp.dot(aggr, wrel_ref[...]) + brel_ref[...] + jnp.dot(g, wroot_ref[...])
        x2 = _lrelu(g2)

        c = _lrelu(jnp.dot(x2, a1w_ref[...]) + a1b_ref[...])
        c = _lrelu(jnp.dot(c, a2w_ref[...]) + a2b_ref[...])
        ids = jnp.dot(c, a3w_ref[...]) + a3b_ref[...]

        p = _lrelu(jnp.dot(x2, c1a_ref[...]) + jnp.dot(ids, c1b_ref[...])
                   + c1bias_ref[...])
        p = _lrelu(jnp.dot(p, c2w_ref[...]) + c2b_ref[...])
        p4 = jnp.dot(p, c3w_ref[...]) + c3b_ref[...]

        out_ref[b, :, 0:6] = ids
        out_ref[b, :, 6:12] = p4
        out_ref[b, :, 12:FO] = jnp.zeros((BIN, FO - 12), jnp.float32)


def _run_bins(phb, Wrel, brel, Wroot, bg, A1, a1, A2, a2, A3, a3,
              C1a, C1b, c1, C2, c2, C3, c3):
    full = lambda s: pl.BlockSpec(s, lambda i: (0, 0))
    return pl.pallas_call(
        _bin_body,
        grid=(NBINS // BPER,),
        in_specs=[
            pl.BlockSpec((BPER, BIN, FH), lambda i: (i, 0, 0)),
            full((32, 32)), full((1, 32)), full((32, 32)), full((1, 32)),
            full((32, 125)), full((1, 125)), full((125, 125)), full((1, 125)),
            full((125, 6)), full((1, 6)),
            full((32, 125)), full((6, 125)), full((1, 125)),
            full((125, 125)), full((1, 125)), full((125, 6)), full((1, 6)),
        ],
        out_specs=pl.BlockSpec((BPER, BIN, FO), lambda i: (i, 0, 0)),
        out_shape=jax.ShapeDtypeStruct((NBINS, BIN, FO), jnp.float32),
        compiler_params=pltpu.CompilerParams(
            dimension_semantics=("arbitrary",)),
    )(phb, Wrel, brel, Wroot, bg, A1, a1, A2, a2, A3, a3,
      C1a, C1b, c1, C2, c2, C3, c3)


_MESH = dict(core_axis_name="c", subcore_axis_name="s")
NW = 32
CH = NP // NW
HF = CH // 2


def _scatter_body(hxw, posh, phxw, pos0, pos1, rows0, rows1, sem):
    wid = lax.axis_index("s") * 2 + lax.axis_index("c")
    base = wid * CH
    for half, (posr, rowsr) in enumerate(((pos0, rows0), (pos1, rows1))):
        b2 = base + half * HF
        pltpu.sync_copy(posh.at[pl.ds(b2, HF)], posr)
        pltpu.sync_copy(hxw.at[pl.ds(b2, HF)], rowsr)
        pltpu.async_copy(rowsr, phxw.at[posr], sem).wait()


def _run_scatter(hxw, pos):
    kern = functools.partial(
        pl.kernel,
        mesh=plsc.VectorSubcoreMesh(**_MESH),
        out_type=jax.ShapeDtypeStruct((NP, FH), jnp.float32),
        scratch_types=[
            pltpu.VMEM((HF,), jnp.int32),
            pltpu.VMEM((HF,), jnp.int32),
            pltpu.VMEM((HF, FH), jnp.float32),
            pltpu.VMEM((HF, FH), jnp.float32),
            pltpu.SemaphoreType.DMA,
        ],
    )(_scatter_body)
    return kern(hxw, pos)


def _gather_body(pres, posh, outh, pos0, pos1, rows0, rows1, sem):
    wid = lax.axis_index("s") * 2 + lax.axis_index("c")
    base = wid * CH
    for half, (posr, rowsr) in enumerate(((pos0, rows0), (pos1, rows1))):
        b2 = base + half * HF
        pltpu.sync_copy(posh.at[pl.ds(b2, HF)], posr)
        pltpu.async_copy(pres.at[posr], rowsr, sem).wait()
        pltpu.sync_copy(rowsr, outh.at[pl.ds(b2, HF)])


def _run_gather(pres, pos):
    kern = functools.partial(
        pl.kernel,
        mesh=plsc.VectorSubcoreMesh(**_MESH),
        out_type=jax.ShapeDtypeStruct((NP, FO), jnp.float32),
        scratch_types=[
            pltpu.VMEM((HF,), jnp.int32),
            pltpu.VMEM((HF,), jnp.int32),
            pltpu.VMEM((HF, FO), jnp.float32),
            pltpu.VMEM((HF, FO), jnp.float32),
            pltpu.SemaphoreType.DMA,
        ],
    )(_gather_body)
    return kern(pres, pos)


def kernel(x, ygen_id, ygen, codebook, W1, b1, W2, b2, W3, b3, Wg, bg, Wrel,
           brel, Wroot, A1, a1, A2, a2, A3, a3, C1, c1, C2, c2, C3, c3):
    row = lambda v: v.reshape(1, -1)
    x_p = jnp.concatenate([x, jnp.zeros((NP - N, 12), x.dtype)], axis=0)

    hxw, binc, rankc, offs = _run_encoder(
        x_p, W1, row(b1), W2, row(b2), W3, row(b3), codebook[:, :25], Wg)

    posc, posgc = _run_pos(binc, rankc, offs)
    pos = posc.reshape(NP)
    posg = posgc.reshape(NP)
    phxw = _run_scatter(hxw, pos)

    phb = phxw[:N].reshape(NBINS, BIN, FH)
    pres = _run_bins(phb, Wrel, row(brel), Wroot, row(bg),
                     A1, row(a1), A2, row(a2), A3, row(a3),
                     C1[:32], C1[32:], row(c1), C2, row(c2), C3, row(c3))

    outp = _run_gather(pres.reshape(N, FO), posg)
    cand_ids = outp[:N, 0:6]
    cand_p4 = outp[:N, 6:12]
    return (cand_ids, cand_p4, ygen_id, ygen)


# speedup vs baseline: 17.7121x; 1.0903x over previous
import functools

import jax
import jax.numpy as jnp
from jax import lax
from jax.experimental import pallas as pl
from jax.experimental.pallas import tpu as pltpu
from jax.experimental.pallas import tpu_sc as plsc

N = 5000
NP = 5120
BLK = 256
NBLK = NP // BLK
NBINS = 50
BIN = 100
K = 16
FH = 128
FO = 128

_NEG_SLOPE = 0.01


def _lrelu(v):
    return jnp.where(v >= 0, v, _NEG_SLOPE * v)


def _enc_body(x_ref, w1_ref, b1_ref, w2_ref, b2_ref, w3_ref, b3_ref, cb_ref,
              wg_ref, hxw_ref, bin_ref, rank_ref, off_ref, carry):
    i = pl.program_id(0)

    @pl.when(i == 0)
    def _init():
        carry[...] = jnp.zeros_like(carry)

    xb = x_ref[...]
    h1 = _lrelu(jnp.dot(xb, w1_ref[...]) + b1_ref[...])
    h2 = _lrelu(jnp.dot(h1, w2_ref[...]) + b2_ref[...])
    h = jnp.dot(h2, w3_ref[...]) + b3_ref[...]

    mul = jnp.dot(h, cb_ref[...])
    io25 = lax.broadcasted_iota(jnp.int32, (BLK, 25), 1)
    mx = jnp.maximum(jnp.max(mul, axis=1, keepdims=True),
                     jnp.max(-mul, axis=1, keepdims=True))
    bpos = jnp.min(jnp.where(mul == mx, io25, 64), axis=1, keepdims=True)
    bneg = jnp.min(jnp.where(-mul == mx, io25 + 25, 64), axis=1, keepdims=True)
    binv = jnp.minimum(bpos, bneg)
    rowid = i * BLK + lax.broadcasted_iota(jnp.int32, (BLK, 1), 0)
    binv = jnp.where(rowid < N, binv, NBINS)

    io64 = lax.broadcasted_iota(jnp.int32, (BLK, 64), 1)
    onehot = (io64 == binv).astype(jnp.float32)
    ior = lax.broadcasted_iota(jnp.int32, (BLK, BLK), 0)
    ioc = lax.broadcasted_iota(jnp.int32, (BLK, BLK), 1)
    lower = (ior > ioc).astype(jnp.float32)
    cum = jnp.dot(lower, onehot)
    carry_now = carry[...]
    rank = jnp.sum(onehot * (carry_now + cum), axis=1, keepdims=True)
    carry[...] = carry_now + jnp.sum(onehot, axis=0, keepdims=True)

    hxw_ref[:, 0:12] = h
    hxw_ref[:, 12:16] = jnp.zeros((BLK, 4), jnp.float32)
    hxw_ref[:, 16:48] = jnp.dot(_lrelu(h), wg_ref[...])
    hxw_ref[:, 48:FH] = jnp.zeros((BLK, FH - 48), jnp.float32)
    bin_ref[...] = binv
    rank_ref[...] = rank.astype(jnp.int32)

    io64r = lax.broadcasted_iota(jnp.int32, (64, 64), 0)
    io64c = lax.broadcasted_iota(jnp.int32, (64, 64), 1)
    strict = (io64r < io64c).astype(jnp.float32)
    off_ref[...] = jnp.dot(carry[...], strict,
                           precision=lax.Precision.HIGHEST).astype(jnp.int32)


def _run_encoder(x_p, W1, b1, W2, b2, W3, b3, cb25, Wg):
    full = lambda s: pl.BlockSpec(s, lambda i: (0, 0))
    return pl.pallas_call(
        _enc_body,
        grid=(NBLK,),
        in_specs=[
            pl.BlockSpec((BLK, 12), lambda i: (i, 0)),
            full((12, 125)), full((1, 125)),
            full((125, 125)), full((1, 125)),
            full((125, 12)), full((1, 12)),
            full((12, 25)), full((12, 32)),
        ],
        out_specs=[
            pl.BlockSpec((BLK, FH), lambda i: (i, 0)),
            pl.BlockSpec((BLK, 1), lambda i: (i, 0)),
            pl.BlockSpec((BLK, 1), lambda i: (i, 0)),
            pl.BlockSpec((1, 64), lambda i: (0, 0)),
        ],
        out_shape=[
            jax.ShapeDtypeStruct((NP, FH), jnp.float32),
            jax.ShapeDtypeStruct((NP, 1), jnp.int32),
            jax.ShapeDtypeStruct((NP, 1), jnp.int32),
            jax.ShapeDtypeStruct((1, 64), jnp.int32),
        ],
        scratch_shapes=[pltpu.VMEM((1, 64), jnp.float32)],
    )(x_p, W1, b1, W2, b2, W3, b3, cb25, Wg)


def _dotT(a, b):
    return lax.dot_general(a, b, (((0,), (0,)), ((), ())),
                           precision=lax.Precision.HIGHEST)


BPER = 10


def _bin_body(ph_ref, wrel_ref, brel_ref, wroot_ref, bg_ref,
              a1w_ref, a1b_ref, a2w_ref, a2b_ref, a3w_ref, a3b_ref,
              c1a_ref, c1b_ref, c1bias_ref, c2w_ref, c2b_ref, c3w_ref, c3b_ref,
              out_ref):
    io = lax.broadcasted_iota(jnp.int32, (BIN, BIN), 1)
    lives, adjs = [], []
    for b in range(BPER):
        pb = ph_ref[b, :, 0:12]
        z = lax.dot_general(pb, pb, (((1,), (1,)), ((), ())))
        lives.append(jax.nn.sigmoid(z))
        adjs.append(jnp.zeros((BIN, BIN), jnp.float32))

    for _ in range(K):
        for b in range(BPER):
            jidx = jnp.argmax(lives[b], axis=1).astype(jnp.int32)
            hit = io == jidx[:, None]
            adjs[b] = jnp.where(hit, lives[b], adjs[b])
            lives[b] = jnp.where(hit, -1.0, lives[b])

    for b in range(BPER):
        adj = adjs[b]
        pxw = ph_ref[b, :, 16:48]
        ones = jnp.ones((BIN, 1), jnp.float32)
        colsum = _dotT(adj, ones)
        dis = lax.rsqrt(colsum + 1.0)
        y = pxw * dis
        t1 = _dotT(adj, y) + y
        g = t1 * dis + bg_ref[...]
        aggr = _dotT(adj, g)
        g2 = jnp.dot(aggr, wrel_ref[...]) + brel_ref[...] + jnp.dot(g, wroot_ref[...])
        x2 = _lrelu(g2)

        c = _lrelu(jnp.dot(x2, a1w_ref[...]) + a1b_ref[...])
        c = _lrelu(jnp.dot(c, a2w_ref[...]) + a2b_ref[...])
        ids = jnp.dot(c, a3w_ref[...]) + a3b_ref[...]

        p = _lrelu(jnp.dot(x2, c1a_ref[...]) + jnp.dot(ids, c1b_ref[...])
                   + c1bias_ref[...])
        p = _lrelu(jnp.dot(p, c2w_ref[...]) + c2b_ref[...])
        p4 = jnp.dot(p, c3w_ref[...]) + c3b_ref[...]

        out_ref[b, :, 0:6] = ids
        out_ref[b, :, 6:12] = p4
        out_ref[b, :, 12:FO] = jnp.zeros((BIN, FO - 12), jnp.float32)


def _run_bins(phb, Wrel, brel, Wroot, bg, A1, a1, A2, a2, A3, a3,
              C1a, C1b, c1, C2, c2, C3, c3):
    full = lambda s: pl.BlockSpec(s, lambda i: (0, 0))
    return pl.pallas_call(
        _bin_body,
        grid=(NBINS // BPER,),
        in_specs=[
            pl.BlockSpec((BPER, BIN, FH), lambda i: (i, 0, 0)),
            full((32, 32)), full((1, 32)), full((32, 32)), full((1, 32)),
            full((32, 125)), full((1, 125)), full((125, 125)), full((1, 125)),
            full((125, 6)), full((1, 6)),
            full((32, 125)), full((6, 125)), full((1, 125)),
            full((125, 125)), full((1, 125)), full((125, 6)), full((1, 6)),
        ],
        out_specs=pl.BlockSpec((BPER, BIN, FO), lambda i: (i, 0, 0)),
        out_shape=jax.ShapeDtypeStruct((NBINS, BIN, FO), jnp.float32),
        compiler_params=pltpu.CompilerParams(
            dimension_semantics=("arbitrary",)),
    )(phb, Wrel, brel, Wroot, bg, A1, a1, A2, a2, A3, a3,
      C1a, C1b, c1, C2, c2, C3, c3)


_MESH = dict(core_axis_name="c", subcore_axis_name="s")
NW = 32
CH = NP // NW
HF = CH // 2


def _scatter_body(hxw, binh, rankh, offh, phxw, posh,
                  off_v, bin_v, rank_v, pos0, pos1, rows0, rows1, sem):
    wid = lax.axis_index("s") * 2 + lax.axis_index("c")
    base = wid * CH
    pltpu.sync_copy(offh, off_v)
    banks = [off_v[pl.ds(g * 16, 16)] for g in range(4)]
    for half, (posr, rowsr) in enumerate(((pos0, rows0), (pos1, rows1))):
        b2 = base + half * HF
        pltpu.sync_copy(binh.at[pl.ds(b2, HF)], bin_v)
        pltpu.sync_copy(rankh.at[pl.ds(b2, HF)], rank_v)
        for j in range(HF // 16):
            sl = pl.ds(j * 16, 16)
            bv = bin_v[sl]
            lo = bv & 15
            bank = bv >> 4
            off = banks[0].at[lo].get(mode="promise_in_bounds")
            for g in range(1, 4):
                og = banks[g].at[lo].get(mode="promise_in_bounds")
                off = jnp.where(bank == g, og, off)
            posr[sl] = off + rank_v[sl]
        pltpu.sync_copy(hxw.at[pl.ds(b2, HF)], rowsr)
        pltpu.async_copy(rowsr, phxw.at[posr], sem).wait()
        pltpu.sync_copy(posr, posh.at[pl.ds(b2, HF)])


def _run_scatter(hxw, binv, rankv, offs):
    kern = functools.partial(
        pl.kernel,
        mesh=plsc.VectorSubcoreMesh(**_MESH),
        out_type=(jax.ShapeDtypeStruct((NP, FH), jnp.float32),
                  jax.ShapeDtypeStruct((NP,), jnp.int32)),
        scratch_types=[
            pltpu.VMEM((64,), jnp.int32),
            pltpu.VMEM((HF,), jnp.int32),
            pltpu.VMEM((HF,), jnp.int32),
            pltpu.VMEM((HF,), jnp.int32),
            pltpu.VMEM((HF,), jnp.int32),
            pltpu.VMEM((HF, FH), jnp.float32),
            pltpu.VMEM((HF, FH), jnp.float32),
            pltpu.SemaphoreType.DMA,
        ],
    )(_scatter_body)
    return kern(hxw, binv, rankv, offs)


def _gather_body(pres, posh, outh, pos0, pos1, rows0, rows1, sem):
    wid = lax.axis_index("s") * 2 + lax.axis_index("c")
    base = wid * CH
    for half, (posr, rowsr) in enumerate(((pos0, rows0), (pos1, rows1))):
        b2 = base + half * HF
        pltpu.sync_copy(posh.at[pl.ds(b2, HF)], posr)
        for j in range(HF // 16):
            sl = pl.ds(j * 16, 16)
            posr[sl] = jnp.minimum(posr[sl], N - 1)
        pltpu.async_copy(pres.at[posr], rowsr, sem).wait()
        pltpu.sync_copy(rowsr, outh.at[pl.ds(b2, HF)])


def _run_gather(pres, pos):
    kern = functools.partial(
        pl.kernel,
        mesh=plsc.VectorSubcoreMesh(**_MESH),
        out_type=jax.ShapeDtypeStruct((NP, FO), jnp.float32),
        scratch_types=[
            pltpu.VMEM((HF,), jnp.int32),
            pltpu.VMEM((HF,), jnp.int32),
            pltpu.VMEM((HF, FO), jnp.float32),
            pltpu.VMEM((HF, FO), jnp.float32),
            pltpu.SemaphoreType.DMA,
        ],
    )(_gather_body)
    return kern(pres, pos)


def kernel(x, ygen_id, ygen, codebook, W1, b1, W2, b2, W3, b3, Wg, bg, Wrel,
           brel, Wroot, A1, a1, A2, a2, A3, a3, C1, c1, C2, c2, C3, c3):
    row = lambda v: v.reshape(1, -1)
    x_p = jnp.concatenate([x, jnp.zeros((NP - N, 12), x.dtype)], axis=0)

    hxw, binc, rankc, offs = _run_encoder(
        x_p, W1, row(b1), W2, row(b2), W3, row(b3), codebook[:, :25], Wg)

    phxw, pos = _run_scatter(hxw, binc.reshape(NP), rankc.reshape(NP),
                             offs.reshape(64))

    phb = phxw[:N].reshape(NBINS, BIN, FH)
    pres = _run_bins(phb, Wrel, row(brel), Wroot, row(bg),
                     A1, row(a1), A2, row(a2), A3, row(a3),
                     C1[:32], C1[32:], row(c1), C2, row(c2), C3, row(c3))

    outp = _run_gather(pres.reshape(N, FO), pos)
    cand_ids = outp[:N, 0:6]
    cand_p4 = outp[:N, 6:12]
    return (cand_ids, cand_p4, ygen_id, ygen)


# speedup vs baseline: 18.8777x; 1.0658x over previous
import functools

import jax
import jax.numpy as jnp
from jax import lax
from jax.experimental import pallas as pl
from jax.experimental.pallas import tpu as pltpu
from jax.experimental.pallas import tpu_sc as plsc

N = 5000
NP = 5120
BLK = 512
NBLK = NP // BLK
NBINS = 50
BIN = 100
K = 16
FH = 128
FO = 128

_NEG_SLOPE = 0.01


def _lrelu(v):
    return jnp.where(v >= 0, v, _NEG_SLOPE * v)


def _enc_body(x_ref, w1_ref, b1_ref, w2_ref, b2_ref, w3_ref, b3_ref, cb_ref,
              wg_ref, hxw_ref, bin_ref, rank_ref, off_ref, carry):
    i = pl.program_id(0)

    @pl.when(i == 0)
    def _init():
        carry[...] = jnp.zeros_like(carry)

    xb = x_ref[...]
    h1 = _lrelu(jnp.dot(xb, w1_ref[...]) + b1_ref[...])
    h2 = _lrelu(jnp.dot(h1, w2_ref[...]) + b2_ref[...])
    h = jnp.dot(h2, w3_ref[...]) + b3_ref[...]

    mul = jnp.dot(h, cb_ref[...])
    io25 = lax.broadcasted_iota(jnp.int32, (BLK, 25), 1)
    mx = jnp.maximum(jnp.max(mul, axis=1, keepdims=True),
                     jnp.max(-mul, axis=1, keepdims=True))
    bpos = jnp.min(jnp.where(mul == mx, io25, 64), axis=1, keepdims=True)
    bneg = jnp.min(jnp.where(-mul == mx, io25 + 25, 64), axis=1, keepdims=True)
    binv = jnp.minimum(bpos, bneg)
    rowid = i * BLK + lax.broadcasted_iota(jnp.int32, (BLK, 1), 0)
    binv = jnp.where(rowid < N, binv, NBINS)

    io64 = lax.broadcasted_iota(jnp.int32, (BLK, 64), 1)
    onehot = (io64 == binv).astype(jnp.float32)
    ior = lax.broadcasted_iota(jnp.int32, (BLK, BLK), 0)
    ioc = lax.broadcasted_iota(jnp.int32, (BLK, BLK), 1)
    lower = (ior > ioc).astype(jnp.float32)
    cum = jnp.dot(lower, onehot)
    carry_now = carry[...]
    rank = jnp.sum(onehot * (carry_now + cum), axis=1, keepdims=True)
    carry[...] = carry_now + jnp.sum(onehot, axis=0, keepdims=True)

    hxw_ref[:, 0:12] = h
    hxw_ref[:, 12:16] = jnp.zeros((BLK, 4), jnp.float32)
    hxw_ref[:, 16:48] = jnp.dot(_lrelu(h), wg_ref[...])
    hxw_ref[:, 48:FH] = jnp.zeros((BLK, FH - 48), jnp.float32)
    bin_ref[...] = binv
    rank_ref[...] = rank.astype(jnp.int32)

    io64r = lax.broadcasted_iota(jnp.int32, (64, 64), 0)
    io64c = lax.broadcasted_iota(jnp.int32, (64, 64), 1)
    strict = (io64r < io64c).astype(jnp.float32)
    off_ref[...] = jnp.dot(carry[...], strict,
                           precision=lax.Precision.HIGHEST).astype(jnp.int32)


def _run_encoder(x_p, W1, b1, W2, b2, W3, b3, cb25, Wg):
    full = lambda s: pl.BlockSpec(s, lambda i: (0, 0))
    return pl.pallas_call(
        _enc_body,
        grid=(NBLK,),
        in_specs=[
            pl.BlockSpec((BLK, 12), lambda i: (i, 0)),
            full((12, 125)), full((1, 125)),
            full((125, 125)), full((1, 125)),
            full((125, 12)), full((1, 12)),
            full((12, 25)), full((12, 32)),
        ],
        out_specs=[
            pl.BlockSpec((BLK, FH), lambda i: (i, 0)),
            pl.BlockSpec((BLK, 1), lambda i: (i, 0)),
            pl.BlockSpec((BLK, 1), lambda i: (i, 0)),
            pl.BlockSpec((1, 64), lambda i: (0, 0)),
        ],
        out_shape=[
            jax.ShapeDtypeStruct((NP, FH), jnp.float32),
            jax.ShapeDtypeStruct((NP, 1), jnp.int32),
            jax.ShapeDtypeStruct((NP, 1), jnp.int32),
            jax.ShapeDtypeStruct((1, 64), jnp.int32),
        ],
        scratch_shapes=[pltpu.VMEM((1, 64), jnp.float32)],
    )(x_p, W1, b1, W2, b2, W3, b3, cb25, Wg)


def _dotT(a, b):
    return lax.dot_general(a, b, (((0,), (0,)), ((), ())),
                           precision=lax.Precision.HIGHEST)


BPER = 25


def _bin_body(ph_ref, wrel_ref, brel_ref, wroot_ref, bg_ref,
              a1w_ref, a1b_ref, a2w_ref, a2b_ref, a3w_ref, a3b_ref,
              c1a_ref, c1b_ref, c1bias_ref, c2w_ref, c2b_ref, c3w_ref, c3b_ref,
              out_ref):
    io = lax.broadcasted_iota(jnp.int32, (BIN, BIN), 1)
    lives, adjs = [], []
    for b in range(BPER):
        pb = ph_ref[b, :, 0:12]
        z = lax.dot_general(pb, pb, (((1,), (1,)), ((), ())))
        lives.append(jax.nn.sigmoid(z))
        adjs.append(jnp.zeros((BIN, BIN), jnp.float32))

    for _ in range(K):
        for b in range(BPER):
            jidx = jnp.argmax(lives[b], axis=1).astype(jnp.int32)
            hit = io == jidx[:, None]
            adjs[b] = jnp.where(hit, lives[b], adjs[b])
            lives[b] = jnp.where(hit, -1.0, lives[b])

    for b in range(BPER):
        adj = adjs[b]
        pxw = ph_ref[b, :, 16:48]
        ones = jnp.ones((BIN, 1), jnp.float32)
        colsum = _dotT(adj, ones)
        dis = lax.rsqrt(colsum + 1.0)
        y = pxw * dis
        t1 = _dotT(adj, y) + y
        g = t1 * dis + bg_ref[...]
        aggr = _dotT(adj, g)
        g2 = jnp.dot(aggr, wrel_ref[...]) + brel_ref[...] + jnp.dot(g, wroot_ref[...])
        x2 = _lrelu(g2)

        c = _lrelu(jnp.dot(x2, a1w_ref[...]) + a1b_ref[...])
        c = _lrelu(jnp.dot(c, a2w_ref[...]) + a2b_ref[...])
        ids = jnp.dot(c, a3w_ref[...]) + a3b_ref[...]

        p = _lrelu(jnp.dot(x2, c1a_ref[...]) + jnp.dot(ids, c1b_ref[...])
                   + c1bias_ref[...])
        p = _lrelu(jnp.dot(p, c2w_ref[...]) + c2b_ref[...])
        p4 = jnp.dot(p, c3w_ref[...]) + c3b_ref[...]

        out_ref[b, :, 0:6] = ids
        out_ref[b, :, 6:12] = p4
        out_ref[b, :, 12:FO] = jnp.zeros((BIN, FO - 12), jnp.float32)


def _run_bins(phb, Wrel, brel, Wroot, bg, A1, a1, A2, a2, A3, a3,
              C1a, C1b, c1, C2, c2, C3, c3):
    full = lambda s: pl.BlockSpec(s, lambda i: (0, 0))
    return pl.pallas_call(
        _bin_body,
        grid=(NBINS // BPER,),
        in_specs=[
            pl.BlockSpec((BPER, BIN, FH), lambda i: (i, 0, 0)),
            full((32, 32)), full((1, 32)), full((32, 32)), full((1, 32)),
            full((32, 125)), full((1, 125)), full((125, 125)), full((1, 125)),
            full((125, 6)), full((1, 6)),
            full((32, 125)), full((6, 125)), full((1, 125)),
            full((125, 125)), full((1, 125)), full((125, 6)), full((1, 6)),
        ],
        out_specs=pl.BlockSpec((BPER, BIN, FO), lambda i: (i, 0, 0)),
        out_shape=jax.ShapeDtypeStruct((NBINS, BIN, FO), jnp.float32),
        compiler_params=pltpu.CompilerParams(
            dimension_semantics=("arbitrary",)),
    )(phb, Wrel, brel, Wroot, bg, A1, a1, A2, a2, A3, a3,
      C1a, C1b, c1, C2, c2, C3, c3)


_MESH = dict(core_axis_name="c", subcore_axis_name="s")
NW = 32
CH = NP // NW
HF = CH // 2


def _scatter_body(hxw, binh, rankh, offh, phxw, posh,
                  off_v, bin_v, rank_v, pos0, pos1, rows0, rows1, sem):
    wid = lax.axis_index("s") * 2 + lax.axis_index("c")
    base = wid * CH
    pltpu.sync_copy(offh, off_v)
    banks = [off_v[pl.ds(g * 16, 16)] for g in range(4)]
    for half, (posr, rowsr) in enumerate(((pos0, rows0), (pos1, rows1))):
        b2 = base + half * HF
        pltpu.sync_copy(binh.at[pl.ds(b2, HF)], bin_v)
        pltpu.sync_copy(rankh.at[pl.ds(b2, HF)], rank_v)
        for j in range(HF // 16):
            sl = pl.ds(j * 16, 16)
            bv = bin_v[sl]
            lo = bv & 15
            bank = bv >> 4
            off = banks[0].at[lo].get(mode="promise_in_bounds")
            for g in range(1, 4):
                og = banks[g].at[lo].get(mode="promise_in_bounds")
                off = jnp.where(bank == g, og, off)
            posr[sl] = off + rank_v[sl]
        pltpu.sync_copy(hxw.at[pl.ds(b2, HF)], rowsr)
        pltpu.async_copy(rowsr, phxw.at[posr], sem).wait()
        pltpu.sync_copy(posr, posh.at[pl.ds(b2, HF)])


def _run_scatter(hxw, binv, rankv, offs):
    kern = functools.partial(
        pl.kernel,
        mesh=plsc.VectorSubcoreMesh(**_MESH),
        out_type=(jax.ShapeDtypeStruct((NP, FH), jnp.float32),
                  jax.ShapeDtypeStruct((NP,), jnp.int32)),
        scratch_types=[
            pltpu.VMEM((64,), jnp.int32),
            pltpu.VMEM((HF,), jnp.int32),
            pltpu.VMEM((HF,), jnp.int32),
            pltpu.VMEM((HF,), jnp.int32),
            pltpu.VMEM((HF,), jnp.int32),
            pltpu.VMEM((HF, FH), jnp.float32),
            pltpu.VMEM((HF, FH), jnp.float32),
            pltpu.SemaphoreType.DMA,
        ],
    )(_scatter_body)
    return kern(hxw, binv, rankv, offs)


def _gather_body(pres, posh, outh, pos0, pos1, rows0, rows1, sem):
    wid = lax.axis_index("s") * 2 + lax.axis_index("c")
    base = wid * CH
    for half, (posr, rowsr) in enumerate(((pos0, rows0), (pos1, rows1))):
        b2 = base + half * HF
        pltpu.sync_copy(posh.at[pl.ds(b2, HF)], posr)
        for j in range(HF // 16):
            sl = pl.ds(j * 16, 16)
            posr[sl] = jnp.minimum(posr[sl], N - 1)
        pltpu.async_copy(pres.at[posr], rowsr, sem).wait()
        pltpu.sync_copy(rowsr, outh.at[pl.ds(b2, HF)])


def _run_gather(pres, pos):
    kern = functools.partial(
        pl.kernel,
        mesh=plsc.VectorSubcoreMesh(**_MESH),
        out_type=jax.ShapeDtypeStruct((NP, FO), jnp.float32),
        scratch_types=[
            pltpu.VMEM((HF,), jnp.int32),
            pltpu.VMEM((HF,), jnp.int32),
            pltpu.VMEM((HF, FO), jnp.float32),
            pltpu.VMEM((HF, FO), jnp.float32),
            pltpu.SemaphoreType.DMA,
        ],
    )(_gather_body)
    return kern(pres, pos)


def kernel(x, ygen_id, ygen, codebook, W1, b1, W2, b2, W3, b3, Wg, bg, Wrel,
           brel, Wroot, A1, a1, A2, a2, A3, a3, C1, c1, C2, c2, C3, c3):
    row = lambda v: v.reshape(1, -1)
    x_p = jnp.concatenate([x, jnp.zeros((NP - N, 12), x.dtype)], axis=0)

    hxw, binc, rankc, offs = _run_encoder(
        x_p, W1, row(b1), W2, row(b2), W3, row(b3), codebook[:, :25], Wg)

    phxw, pos = _run_scatter(hxw, binc.reshape(NP), rankc.reshape(NP),
                             offs.reshape(64))

    phb = phxw[:N].reshape(NBINS, BIN, FH)
    pres = _run_bins(phb, Wrel, row(brel), Wroot, row(bg),
                     A1, row(a1), A2, row(a2), A3, row(a3),
                     C1[:32], C1[32:], row(c1), C2, row(c2), C3, row(c3))

    outp = _run_gather(pres.reshape(N, FO), pos)
    cand_ids = outp[:N, 0:6]
    cand_p4 = outp[:N, 6:12]
    return (cand_ids, cand_p4, ygen_id, ygen)
